# baseline (device time: 3963639 ns/iter reference)
import jax
import jax.numpy as jnp
from jax import lax
from jax.experimental import pallas as pl
from jax.experimental.pallas import tpu as pltpu

N_DEV = 4
NC = 4
N_RS = (N_DEV - 1) * NC
N_IT = 2 * N_RS


def _gelu(y):
    c = 0.7978845608028654
    return 0.5 * y * (1.0 + jnp.tanh(c * (y + 0.044715 * y * y * y)))


def kernel(x, w_mat):
    hn = w_mat.shape[1] // 2
    p_l = jnp.dot(x, w_mat[:, :hn], preferred_element_type=jnp.float32)
    p_r = jnp.dot(x, w_mat[:, hn:], preferred_element_type=jnp.float32)
    return _allreduce_gelu(p_l, p_r)


def _allreduce_gelu(p_l, p_r):
    M, HN = p_l.shape
    N = 2 * HN
    BLK_M = M // N_DEV
    CH_M = BLK_M // NC

    def body(pl_ref, pr_ref, out_ref, outl_ref, outr_ref,
             acc_r_hbm, acc_l_hbm, recv_r_hbm, recv_l_hbm, a_buf, b_buf,
             send_sems_r, recv_sems_r, send_sems_l, recv_sems_l,
             credit_r, credit_l, cp_sems, wb_sems):
        my = lax.axis_index("i")
        left = (my + N_DEV - 1) % N_DEV
        right = (my + 1) % N_DEV

        barrier = pltpu.get_barrier_semaphore()
        for nbr in (left, right):
            pl.semaphore_signal(barrier, inc=1, device_id=(nbr,),
                                device_id_type=pl.DeviceIdType.MESH)
        pl.semaphore_wait(barrier, 2)

        def make_rdma(it):
            slot = it % 2
            if it < N_RS:
                h, c = divmod(it, NC)
                ro = c * CH_M
                sb_r = (my + N_DEV - h) % N_DEV
                sb_l = (my + h) % N_DEV
                if h == 0:
                    src_r = pl_ref.at[pl.ds(sb_r * BLK_M + ro, CH_M), :]
                    src_l = pr_ref.at[pl.ds(sb_l * BLK_M + ro, CH_M), :]
                else:
                    src_r = acc_r_hbm.at[(h - 1) % 2, pl.ds(ro, CH_M), :]
                    src_l = acc_l_hbm.at[(h - 1) % 2, pl.ds(ro, CH_M), :]
                dst_r = recv_r_hbm.at[slot]
                dst_l = recv_l_hbm.at[slot]
            else:
                g, c = divmod(it - N_RS, NC)
                ro = c * CH_M
                gs_r = (my + N_DEV + 1 - g) % N_DEV
                gs_l = (my + N_DEV - 1 + g) % N_DEV
                src_r = dst_r = outl_ref.at[pl.ds(gs_r * BLK_M + ro, CH_M), :]
                src_l = dst_l = outr_ref.at[pl.ds(gs_l * BLK_M + ro, CH_M), :]
            rdma_r = pltpu.make_async_remote_copy(
                src_ref=src_r, dst_ref=dst_r,
                send_sem=send_sems_r.at[slot], recv_sem=recv_sems_r.at[slot],
                device_id=(right,), device_id_type=pl.DeviceIdType.MESH)
            rdma_l = pltpu.make_async_remote_copy(
                src_ref=src_l, dst_ref=dst_l,
                send_sem=send_sems_l.at[slot], recv_sem=recv_sems_l.at[slot],
                device_id=(left,), device_id_type=pl.DeviceIdType.MESH)
            return rdma_r, rdma_l

        descs = {}

        def start(it):
            if it >= 2:
                descs[it - 2][0].wait_send()
                descs[it - 2][1].wait_send()
                pl.semaphore_wait(credit_r, 1)
                pl.semaphore_wait(credit_l, 1)
            d = make_rdma(it)
            descs[it] = d
            d[0].start()
            d[1].start()

        def consume(it):
            slot = it % 2
            if it < N_RS:
                h, c = divmod(it, NC)
                ro = c * CH_M
                rb_r = (my + N_DEV - h - 1) % N_DEV
                rb_l = (my + h + 1) % N_DEV
                cps = [
                    pltpu.make_async_copy(
                        recv_r_hbm.at[slot], a_buf.at[:, pl.ds(0, HN)],
                        cp_sems.at[0]),
                    pltpu.make_async_copy(
                        recv_l_hbm.at[slot], a_buf.at[:, pl.ds(HN, HN)],
                        cp_sems.at[1]),
                    pltpu.make_async_copy(
                        pl_ref.at[pl.ds(rb_r * BLK_M + ro, CH_M), :],
                        b_buf.at[:, pl.ds(0, HN)], cp_sems.at[2]),
                    pltpu.make_async_copy(
                        pr_ref.at[pl.ds(rb_l * BLK_M + ro, CH_M), :],
                        b_buf.at[:, pl.ds(HN, HN)], cp_sems.at[3]),
                ]
                for cp in cps:
                    cp.start()
                for cp in cps:
                    cp.wait()
                pl.semaphore_signal(credit_r, inc=1, device_id=(left,),
                                    device_id_type=pl.DeviceIdType.MESH)
                pl.semaphore_signal(credit_l, inc=1, device_id=(right,),
                                    device_id_type=pl.DeviceIdType.MESH)
                if h < N_DEV - 2:
                    a_buf[:, :] = a_buf[:, :] + b_buf[:, :]
                    wbs = [
                        pltpu.make_async_copy(
                            a_buf.at[:, pl.ds(0, HN)],
                            acc_r_hbm.at[h % 2, pl.ds(ro, CH_M), :],
                            wb_sems.at[0]),
                        pltpu.make_async_copy(
                            a_buf.at[:, pl.ds(HN, HN)],
                            acc_l_hbm.at[h % 2, pl.ds(ro, CH_M), :],
                            wb_sems.at[1]),
                    ]
                else:
                    a_buf[:, :] = _gelu(a_buf[:, :] + b_buf[:, :])
                    wbs = [
                        pltpu.make_async_copy(
                            a_buf.at[:, pl.ds(0, HN)],
                            outl_ref.at[pl.ds(rb_r * BLK_M + ro, CH_M), :],
                            wb_sems.at[0]),
                        pltpu.make_async_copy(
                            a_buf.at[:, pl.ds(HN, HN)],
                            outr_ref.at[pl.ds(rb_l * BLK_M + ro, CH_M), :],
                            wb_sems.at[1]),
                        pltpu.make_async_copy(
                            a_buf.at[:, pl.ds(0, HN)],
                            out_ref.at[pl.ds(rb_r * BLK_M + ro, CH_M),
                                       pl.ds(0, HN)],
                            wb_sems.at[2]),
                        pltpu.make_async_copy(
                            a_buf.at[:, pl.ds(HN, HN)],
                            out_ref.at[pl.ds(rb_l * BLK_M + ro, CH_M),
                                       pl.ds(HN, HN)],
                            wb_sems.at[3]),
                    ]
                for wb in wbs:
                    wb.start()
                for wb in wbs:
                    wb.wait()
            else:
                g, c = divmod(it - N_RS, NC)
                ro = c * CH_M
                pl.semaphore_signal(credit_r, inc=1, device_id=(left,),
                                    device_id_type=pl.DeviceIdType.MESH)
                pl.semaphore_signal(credit_l, inc=1, device_id=(right,),
                                    device_id_type=pl.DeviceIdType.MESH)
                gr_r = (my + N_DEV - g) % N_DEV
                gr_l = (my + g) % N_DEV
                asm = [
                    pltpu.make_async_copy(
                        outl_ref.at[pl.ds(gr_r * BLK_M + ro, CH_M), :],
                        out_ref.at[pl.ds(gr_r * BLK_M + ro, CH_M),
                                   pl.ds(0, HN)],
                        wb_sems.at[0]),
                    pltpu.make_async_copy(
                        outr_ref.at[pl.ds(gr_l * BLK_M + ro, CH_M), :],
                        out_ref.at[pl.ds(gr_l * BLK_M + ro, CH_M),
                                   pl.ds(HN, HN)],
                        wb_sems.at[1]),
                ]
                for cp in asm:
                    cp.start()
                for cp in asm:
                    cp.wait()

        start(0)
        for it in range(N_IT):
            descs[it][0].wait_recv()
            descs[it][1].wait_recv()
            if it + 1 < N_IT:
                start(it + 1)
            consume(it)

        for it in (N_IT - 2, N_IT - 1):
            descs[it][0].wait_send()
            descs[it][1].wait_send()
        pl.semaphore_wait(credit_r, 2)
        pl.semaphore_wait(credit_l, 2)

    out, *_ = pl.pallas_call(
        body,
        out_shape=[
            jax.ShapeDtypeStruct((M, N), jnp.float32),
            jax.ShapeDtypeStruct((M, HN), jnp.float32),
            jax.ShapeDtypeStruct((M, HN), jnp.float32),
            jax.ShapeDtypeStruct((2, BLK_M, HN), jnp.float32),
            jax.ShapeDtypeStruct((2, BLK_M, HN), jnp.float32),
            jax.ShapeDtypeStruct((2, CH_M, HN), jnp.float32),
            jax.ShapeDtypeStruct((2, CH_M, HN), jnp.float32),
        ],
        in_specs=[pl.BlockSpec(memory_space=pl.ANY)] * 2,
        out_specs=[pl.BlockSpec(memory_space=pl.ANY)] * 7,
        scratch_shapes=[
            pltpu.VMEM((CH_M, N), jnp.float32),
            pltpu.VMEM((CH_M, N), jnp.float32),
            pltpu.SemaphoreType.DMA((2,)),
            pltpu.SemaphoreType.DMA((2,)),
            pltpu.SemaphoreType.DMA((2,)),
            pltpu.SemaphoreType.DMA((2,)),
            pltpu.SemaphoreType.REGULAR,
            pltpu.SemaphoreType.REGULAR,
            pltpu.SemaphoreType.DMA((4,)),
            pltpu.SemaphoreType.DMA((4,)),
        ],
        compiler_params=pltpu.CompilerParams(collective_id=0),
    )(p_l, p_r)
    return out


# device time: 1410788 ns/iter; 2.8095x vs baseline; 2.8095x over previous
import jax
import jax.numpy as jnp
from jax import lax
from jax.experimental import pallas as pl
from jax.experimental.pallas import tpu as pltpu

N_DEV = 4
NC = 4
N_RS = (N_DEV - 1) * NC
N_IT = 2 * N_RS


def _gelu(y):
    c = 0.7978845608028654
    return 0.5 * y * (1.0 + jnp.tanh(c * (y + 0.044715 * y * y * y)))


def kernel(x, w_mat):
    hn = w_mat.shape[1] // 2
    p_l = jnp.dot(x, w_mat[:, :hn], preferred_element_type=jnp.float32)
    p_r = jnp.dot(x, w_mat[:, hn:], preferred_element_type=jnp.float32)
    out_l, out_r = _allreduce_gelu(p_l, p_r)
    return jnp.concatenate([out_l, out_r], axis=1)


def _allreduce_gelu(p_l, p_r):
    M, HN = p_l.shape
    N = 2 * HN
    BLK_M = M // N_DEV
    CH_M = BLK_M // NC

    def body(pl_ref, pr_ref, outl_ref, outr_ref,
             acc_r_hbm, acc_l_hbm, recv_r_hbm, recv_l_hbm, a_buf, b_buf,
             send_sems_r, recv_sems_r, send_sems_l, recv_sems_l,
             credit_r, credit_l, cp_sems, wb_sems):
        my = lax.axis_index("i")
        left = (my + N_DEV - 1) % N_DEV
        right = (my + 1) % N_DEV

        barrier = pltpu.get_barrier_semaphore()
        for nbr in (left, right):
            pl.semaphore_signal(barrier, inc=1, device_id=(nbr,),
                                device_id_type=pl.DeviceIdType.MESH)
        pl.semaphore_wait(barrier, 2)

        def make_rdma(it):
            slot = it % 2
            if it < N_RS:
                h, c = divmod(it, NC)
                ro = c * CH_M
                sb_r = (my + N_DEV - h) % N_DEV
                sb_l = (my + h) % N_DEV
                if h == 0:
                    src_r = pl_ref.at[pl.ds(sb_r * BLK_M + ro, CH_M), :]
                    src_l = pr_ref.at[pl.ds(sb_l * BLK_M + ro, CH_M), :]
                else:
                    src_r = acc_r_hbm.at[(h - 1) % 2, pl.ds(ro, CH_M), :]
                    src_l = acc_l_hbm.at[(h - 1) % 2, pl.ds(ro, CH_M), :]
                dst_r = recv_r_hbm.at[slot]
                dst_l = recv_l_hbm.at[slot]
            else:
                g, c = divmod(it - N_RS, NC)
                ro = c * CH_M
                gs_r = (my + N_DEV + 1 - g) % N_DEV
                gs_l = (my + N_DEV - 1 + g) % N_DEV
                src_r = dst_r = outl_ref.at[pl.ds(gs_r * BLK_M + ro, CH_M), :]
                src_l = dst_l = outr_ref.at[pl.ds(gs_l * BLK_M + ro, CH_M), :]
            rdma_r = pltpu.make_async_remote_copy(
                src_ref=src_r, dst_ref=dst_r,
                send_sem=send_sems_r.at[slot], recv_sem=recv_sems_r.at[slot],
                device_id=(right,), device_id_type=pl.DeviceIdType.MESH)
            rdma_l = pltpu.make_async_remote_copy(
                src_ref=src_l, dst_ref=dst_l,
                send_sem=send_sems_l.at[slot], recv_sem=recv_sems_l.at[slot],
                device_id=(left,), device_id_type=pl.DeviceIdType.MESH)
            return rdma_r, rdma_l

        descs = {}

        def start(it):
            if it >= 2:
                descs[it - 2][0].wait_send()
                descs[it - 2][1].wait_send()
                pl.semaphore_wait(credit_r, 1)
                pl.semaphore_wait(credit_l, 1)
            d = make_rdma(it)
            descs[it] = d
            d[0].start()
            d[1].start()

        def consume(it):
            slot = it % 2
            if it < N_RS:
                h, c = divmod(it, NC)
                ro = c * CH_M
                rb_r = (my + N_DEV - h - 1) % N_DEV
                rb_l = (my + h + 1) % N_DEV
                cps = [
                    pltpu.make_async_copy(
                        recv_r_hbm.at[slot], a_buf.at[:, pl.ds(0, HN)],
                        cp_sems.at[0]),
                    pltpu.make_async_copy(
                        recv_l_hbm.at[slot], a_buf.at[:, pl.ds(HN, HN)],
                        cp_sems.at[1]),
                    pltpu.make_async_copy(
                        pl_ref.at[pl.ds(rb_r * BLK_M + ro, CH_M), :],
                        b_buf.at[:, pl.ds(0, HN)], cp_sems.at[2]),
                    pltpu.make_async_copy(
                        pr_ref.at[pl.ds(rb_l * BLK_M + ro, CH_M), :],
                        b_buf.at[:, pl.ds(HN, HN)], cp_sems.at[3]),
                ]
                for cp in cps:
                    cp.start()
                for cp in cps:
                    cp.wait()
                pl.semaphore_signal(credit_r, inc=1, device_id=(left,),
                                    device_id_type=pl.DeviceIdType.MESH)
                pl.semaphore_signal(credit_l, inc=1, device_id=(right,),
                                    device_id_type=pl.DeviceIdType.MESH)
                if h < N_DEV - 2:
                    a_buf[:, :] = a_buf[:, :] + b_buf[:, :]
                    wbs = [
                        pltpu.make_async_copy(
                            a_buf.at[:, pl.ds(0, HN)],
                            acc_r_hbm.at[h % 2, pl.ds(ro, CH_M), :],
                            wb_sems.at[0]),
                        pltpu.make_async_copy(
                            a_buf.at[:, pl.ds(HN, HN)],
                            acc_l_hbm.at[h % 2, pl.ds(ro, CH_M), :],
                            wb_sems.at[1]),
                    ]
                else:
                    a_buf[:, :] = _gelu(a_buf[:, :] + b_buf[:, :])
                    wbs = [
                        pltpu.make_async_copy(
                            a_buf.at[:, pl.ds(0, HN)],
                            outl_ref.at[pl.ds(rb_r * BLK_M + ro, CH_M), :],
                            wb_sems.at[0]),
                        pltpu.make_async_copy(
                            a_buf.at[:, pl.ds(HN, HN)],
                            outr_ref.at[pl.ds(rb_l * BLK_M + ro, CH_M), :],
                            wb_sems.at[1]),
                    ]
                for wb in wbs:
                    wb.start()
                for wb in wbs:
                    wb.wait()
            else:
                pl.semaphore_signal(credit_r, inc=1, device_id=(left,),
                                    device_id_type=pl.DeviceIdType.MESH)
                pl.semaphore_signal(credit_l, inc=1, device_id=(right,),
                                    device_id_type=pl.DeviceIdType.MESH)

        start(0)
        for it in range(N_IT):
            descs[it][0].wait_recv()
            descs[it][1].wait_recv()
            if it + 1 < N_IT:
                start(it + 1)
            consume(it)

        for it in (N_IT - 2, N_IT - 1):
            descs[it][0].wait_send()
            descs[it][1].wait_send()
        pl.semaphore_wait(credit_r, 2)
        pl.semaphore_wait(credit_l, 2)

    out_l, out_r, *_ = pl.pallas_call(
        body,
        out_shape=[
            jax.ShapeDtypeStruct((M, HN), jnp.float32),
            jax.ShapeDtypeStruct((M, HN), jnp.float32),
            jax.ShapeDtypeStruct((2, BLK_M, HN), jnp.float32),
            jax.ShapeDtypeStruct((2, BLK_M, HN), jnp.float32),
            jax.ShapeDtypeStruct((2, CH_M, HN), jnp.float32),
            jax.ShapeDtypeStruct((2, CH_M, HN), jnp.float32),
        ],
        in_specs=[pl.BlockSpec(memory_space=pl.ANY)] * 2,
        out_specs=[pl.BlockSpec(memory_space=pl.ANY)] * 6,
        scratch_shapes=[
            pltpu.VMEM((CH_M, N), jnp.float32),
            pltpu.VMEM((CH_M, N), jnp.float32),
            pltpu.SemaphoreType.DMA((2,)),
            pltpu.SemaphoreType.DMA((2,)),
            pltpu.SemaphoreType.DMA((2,)),
            pltpu.SemaphoreType.DMA((2,)),
            pltpu.SemaphoreType.REGULAR,
            pltpu.SemaphoreType.REGULAR,
            pltpu.SemaphoreType.DMA((4,)),
            pltpu.SemaphoreType.DMA((4,)),
        ],
        compiler_params=pltpu.CompilerParams(collective_id=0),
    )(p_l, p_r)
    return out_l, out_r
